# Initial kernel scaffold; baseline (speedup 1.0000x reference)
import functools
import jax
import jax.numpy as jnp
from jax import lax
from jax.experimental import pallas as pl
from jax.experimental.pallas import tpu as pltpu
from jax.experimental.pallas import tpu_sc as plsc

N = 10000
E = 320000
DH = 128
NC = 2
NS = 16
K = 80
NCHUNK_AGG = (E // NS) // K
NCHUNK_DEG = (E // (NS * NC)) // K
ROWS_PT = N // NS
ZR = 125

_mesh = plsc.VectorSubcoreMesh(core_axis_name="c", subcore_axis_name="s")


def _deg_body(dst_hbm, out_hbm, dst_v, obuf, zbuf, acc):
    c = lax.axis_index("c")
    s = lax.axis_index("s")
    ones16 = jnp.ones((16,), jnp.float32)
    zeros16 = jnp.zeros((16,), jnp.float32)

    def fill_ones(r, carry):
        obuf[r, :] = ones16
        return carry
    lax.fori_loop(0, K, fill_ones, 0)

    def fill_zeros(r, carry):
        zbuf[r, :] = zeros16
        return carry
    lax.fori_loop(0, ZR, fill_zeros, 0)

    row0 = s * ROWS_PT
    for q in range(ROWS_PT // ZR):
        pltpu.sync_copy(zbuf, acc.at[pl.ds(row0 + q * ZR, ZR)])

    pltpu.sync_copy(dst_hbm.at[s, pl.ds(c * NCHUNK_DEG, NCHUNK_DEG)], dst_v)
    plsc.subcore_barrier()

    def step(j, carry):
        pltpu.sync_copy(obuf, acc.at[dst_v.at[j]], add=True)
        return carry
    lax.fori_loop(0, NCHUNK_DEG, step, 0)

    plsc.subcore_barrier()
    pltpu.sync_copy(acc.at[pl.ds(row0, ROWS_PT)],
                    out_hbm.at[pl.ds(c * N + row0, ROWS_PT)])


_deg_call = functools.partial(
    pl.kernel,
    out_type=jax.ShapeDtypeStruct((NC * N, 16), jnp.float32),
    mesh=_mesh,
    scratch_types=[
        pltpu.VMEM((NCHUNK_DEG, K), jnp.int32),
        pltpu.VMEM((K, 16), jnp.float32),
        pltpu.VMEM((ZR, 16), jnp.float32),
        pltpu.VMEM_SHARED((N, 16), jnp.float32),
    ],
)(_deg_body)


def _agg_body(src_hbm, dst_hbm, g_hbm, out_hbm, src_v, dst_v, buf, zbuf, acc,
              gsem0, gsem1, ssem0, ssem1):
    c = lax.axis_index("c")
    s = lax.axis_index("s")
    zeros16 = jnp.zeros((16,), jnp.float32)

    def zrow(r, carry):
        for k in range(DH // 16):
            zbuf[r, pl.ds(k * 16, 16)] = zeros16
        return carry
    lax.fori_loop(0, ZR, zrow, 0)

    row0 = s * ROWS_PT
    for q in range(ROWS_PT // ZR):
        pltpu.sync_copy(zbuf, acc.at[pl.ds(row0 + q * ZR, ZR)])

    pltpu.sync_copy(src_hbm.at[s], src_v)
    pltpu.sync_copy(dst_hbm.at[s], dst_v)

    off = c * N

    def addoff(r, carry):
        for k in range(K // 16):
            sl = pl.ds(k * 16, 16)
            src_v[r, sl] = src_v[r, sl] + off
        return carry
    lax.fori_loop(0, NCHUNK_AGG, addoff, 0)

    plsc.subcore_barrier()

    def gather(j, b, sem):
        pltpu.async_copy(g_hbm.at[src_v.at[j]], buf.at[b], sem)

    gather(0, 0, gsem0)
    last = NCHUNK_AGG - 1

    def step(i, carry):
        j0 = i * 2
        pltpu.make_async_copy(g_hbm.at[src_v.at[j0]], buf.at[0], gsem0).wait()
        gather(jnp.minimum(j0 + 1, last), 1, gsem1)
        pltpu.async_copy(buf.at[0], acc.at[dst_v.at[j0]], ssem0, add=True)
        pltpu.make_async_copy(buf.at[0], acc.at[dst_v.at[j0]], ssem0).wait()
        j1 = j0 + 1
        pltpu.make_async_copy(g_hbm.at[src_v.at[j1]], buf.at[1], gsem1).wait()
        gather(jnp.minimum(j0 + 2, last), 0, gsem0)
        pltpu.async_copy(buf.at[1], acc.at[dst_v.at[j1]], ssem1, add=True)
        pltpu.make_async_copy(buf.at[1], acc.at[dst_v.at[j1]], ssem1).wait()
        return carry
    lax.fori_loop(0, NCHUNK_AGG // 2, step, 0)

    pltpu.make_async_copy(g_hbm.at[src_v.at[0]], buf.at[0], gsem0).wait()

    plsc.subcore_barrier()
    pltpu.sync_copy(acc.at[pl.ds(row0, ROWS_PT)],
                    out_hbm.at[pl.ds(c * N + row0, ROWS_PT)])


_agg_call = functools.partial(
    pl.kernel,
    out_type=jax.ShapeDtypeStruct((NC * N, DH), jnp.float32),
    mesh=_mesh,
    scratch_types=[
        pltpu.VMEM((NCHUNK_AGG, K), jnp.int32),
        pltpu.VMEM((NCHUNK_AGG, K), jnp.int32),
        pltpu.VMEM((2, K, DH), jnp.float32),
        pltpu.VMEM((ZR, DH), jnp.float32),
        pltpu.VMEM_SHARED((N, DH), jnp.float32),
        pltpu.SemaphoreType.DMA,
        pltpu.SemaphoreType.DMA,
        pltpu.SemaphoreType.DMA,
        pltpu.SemaphoreType.DMA,
    ],
)(_agg_body)


R = 500
NB = N // R


def _dinv_block(deg_ref):
    d = deg_ref[0, :, 0:1] + deg_ref[1, :, 0:1] + 1.0
    return jnp.where(d > 0, lax.rsqrt(jnp.maximum(d, 1e-12)), 0.0)


def _mm1_body(deg_ref, x_ref, w_ref, o_ref):
    dinv = _dinv_block(deg_ref)
    h = jnp.dot(x_ref[...], w_ref[...], preferred_element_type=jnp.float32)
    g = h * dinv
    o_ref[0] = g[:, :DH]
    o_ref[1] = g[:, DH:]


def _layer_body(deg_ref, agg_ref, g_ref, b_ref, w_ref, o_ref):
    dinv = _dinv_block(deg_ref)
    x0 = jnp.maximum((agg_ref[0] + g_ref[0]) * dinv + b_ref[0, :DH], 0.0)
    x1 = jnp.maximum((agg_ref[1] + g_ref[1]) * dinv + b_ref[0, DH:], 0.0)
    x = jnp.concatenate([x0, x1], axis=1)
    h = jnp.dot(x, w_ref[...], preferred_element_type=jnp.float32)
    g = h * dinv
    o_ref[0] = g[:, :DH]
    o_ref[1] = g[:, DH:]


def _final_body(deg_ref, agg_ref, g_ref, b_ref, o_ref):
    dinv = _dinv_block(deg_ref)
    x0 = (agg_ref[0] + g_ref[0]) * dinv + b_ref[0, :DH]
    x1 = (agg_ref[1] + g_ref[1]) * dinv + b_ref[0, DH:]
    o_ref[...] = jnp.concatenate([x0, x1], axis=1)


_deg_spec = pl.BlockSpec((2, R, 16), lambda j: (0, j, 0))
_half_spec = pl.BlockSpec((2, R, DH), lambda j: (0, j, 0))
_b_spec = pl.BlockSpec((1, 2 * DH), lambda j: (0, 0))


def _mm1(deg2, x, W1):
    return pl.pallas_call(
        _mm1_body,
        grid=(NB,),
        in_specs=[
            _deg_spec,
            pl.BlockSpec((R, 128), lambda j: (j, 0)),
            pl.BlockSpec((128, 2 * DH), lambda j: (0, 0)),
        ],
        out_specs=_half_spec,
        out_shape=jax.ShapeDtypeStruct((2, N, DH), jnp.float32),
    )(deg2, x, W1)


def _layer(deg2, agg, g, b, W):
    return pl.pallas_call(
        _layer_body,
        grid=(NB,),
        in_specs=[
            _deg_spec,
            _half_spec,
            _half_spec,
            _b_spec,
            pl.BlockSpec((2 * DH, 2 * DH), lambda j: (0, 0)),
        ],
        out_specs=_half_spec,
        out_shape=jax.ShapeDtypeStruct((2, N, DH), jnp.float32),
    )(deg2, agg, g, b, W)


def _final(deg2, agg, g, b):
    return pl.pallas_call(
        _final_body,
        grid=(NB,),
        in_specs=[_deg_spec, _half_spec, _half_spec, _b_spec],
        out_specs=pl.BlockSpec((R, 2 * DH), lambda j: (j, 0)),
        out_shape=jax.ShapeDtypeStruct((N, 2 * DH), jnp.float32),
    )(deg2, agg, g, b)


def kernel(x, edge_index, W1, b1, W2, b2, W3, b3):
    src_r = edge_index[0].reshape(NS, NCHUNK_AGG, K)
    dst_r = edge_index[1].reshape(NS, NCHUNK_AGG, K)
    b1r = b1.reshape(1, 2 * DH)
    b2r = b2.reshape(1, 2 * DH)
    b3r = b3.reshape(1, 2 * DH)

    deg2 = _deg_call(dst_r).reshape(2, N, 16)

    g1 = _mm1(deg2, x, W1)
    agg1 = _agg_call(src_r, dst_r, g1.reshape(NC * N, DH)).reshape(2, N, DH)
    g2 = _layer(deg2, agg1, g1, b1r, W2)
    agg2 = _agg_call(src_r, dst_r, g2.reshape(NC * N, DH)).reshape(2, N, DH)
    g3 = _layer(deg2, agg2, g2, b2r, W3)
    agg3 = _agg_call(src_r, dst_r, g3.reshape(NC * N, DH)).reshape(2, N, DH)
    return _final(deg2, agg3, g3, b3r)


# speedup vs baseline: 10.8985x; 10.8985x over previous
import functools
import jax
import jax.numpy as jnp
from jax import lax
from jax.experimental import pallas as pl
from jax.experimental.pallas import tpu as pltpu
from jax.experimental.pallas import tpu_sc as plsc

N = 10000
E = 320000
DH = 128
NC = 2
NS = 16
K = 80
NCHUNK_AGG = (E // NS) // K
NCHUNK_DEG = (E // (NS * NC)) // K
SB = 25
NSB_AGG = NCHUNK_AGG // SB
NSB_DEG = NCHUNK_DEG // SB
NPAD = 10240
ROWS_PT = NPAD // NS
ZR = 128

_mesh = plsc.VectorSubcoreMesh(core_axis_name="c", subcore_axis_name="s")


def _deg_body(dst_hbm, out_hbm, dst_v, obuf, acc):
    c = lax.axis_index("c")
    s = lax.axis_index("s")
    ones16 = jnp.ones((16,), jnp.float32)
    zeros16 = jnp.zeros((16,), jnp.float32)

    def fill(val):
        def body(r, carry):
            obuf[r, :] = val
            return carry
        lax.fori_loop(0, K, body, 0)

    fill(zeros16)
    row0 = s * ROWS_PT
    for q in range(ROWS_PT // K):
        pltpu.sync_copy(obuf, acc.at[pl.ds(row0 + q * K, K)])
    fill(ones16)
    plsc.subcore_barrier()

    wid = s * NC + c

    def sblock(sb, carry):
        pltpu.sync_copy(dst_hbm.at[wid, sb], dst_v)

        def step(j, carry2):
            pltpu.sync_copy(obuf, acc.at[dst_v.at[j]], add=True)
            return carry2
        lax.fori_loop(0, SB, step, 0)
        return carry
    lax.fori_loop(0, NSB_DEG, sblock, 0)

    plsc.subcore_barrier()
    pltpu.sync_copy(acc.at[pl.ds(row0, ROWS_PT)],
                    out_hbm.at[pl.ds(c * NPAD + row0, ROWS_PT)])


_deg_call = functools.partial(
    pl.kernel,
    out_type=jax.ShapeDtypeStruct((NC * NPAD, 16), jnp.float32),
    mesh=_mesh,
    scratch_types=[
        pltpu.VMEM((SB, K), jnp.int32),
        pltpu.VMEM((K, 16), jnp.float32),
        pltpu.VMEM_SHARED((NPAD, 16), jnp.float32),
    ],
)(_deg_body)


def _agg_body(src_hbm, dst_hbm, g_hbm, out_hbm, src_v, dst_v, buf, acc,
              gsem0, gsem1, ssem0, ssem1):
    c = lax.axis_index("c")
    s = lax.axis_index("s")
    zeros16 = jnp.zeros((16,), jnp.float32)

    def zrow(r, carry):
        for k in range(DH // 16):
            buf[0, r, pl.ds(k * 16, 16)] = zeros16
        return carry
    lax.fori_loop(0, K, zrow, 0)

    row0 = s * ROWS_PT
    for q in range(ROWS_PT // K):
        pltpu.sync_copy(buf.at[0], acc.at[pl.ds(row0 + q * K, K)])

    off = c * NPAD
    plsc.subcore_barrier()

    def gather(j, b, sem):
        pltpu.async_copy(g_hbm.at[src_v.at[j]], buf.at[b], sem)

    def gwait(j, b, sem):
        pltpu.make_async_copy(g_hbm.at[src_v.at[j]], buf.at[b], sem).wait()

    def scatter(j, b, sem):
        pltpu.async_copy(buf.at[b], acc.at[dst_v.at[j]], sem, add=True)
        pltpu.make_async_copy(buf.at[b], acc.at[dst_v.at[j]], sem).wait()

    def sblock(sb, carry):
        pltpu.sync_copy(src_hbm.at[s, sb], src_v)
        pltpu.sync_copy(dst_hbm.at[s, sb], dst_v)

        def addoff(r, carry2):
            for k in range(K // 16):
                sl = pl.ds(k * 16, 16)
                src_v[r, sl] = src_v[r, sl] + off
            return carry2
        lax.fori_loop(0, SB, addoff, 0)

        gather(0, 0, gsem0)

        def pair(i, carry2):
            j0 = i * 2
            gwait(j0, 0, gsem0)
            gather(j0 + 1, 1, gsem1)
            scatter(j0, 0, ssem0)
            gwait(j0 + 1, 1, gsem1)
            gather(jnp.minimum(j0 + 2, SB - 1), 0, gsem0)
            scatter(j0 + 1, 1, ssem1)
            return carry2
        lax.fori_loop(0, SB // 2, pair, 0)

        gwait(SB - 1, 0, gsem0)
        scatter(SB - 1, 0, ssem0)
        return carry
    lax.fori_loop(0, NSB_AGG, sblock, 0)

    plsc.subcore_barrier()
    pltpu.sync_copy(acc.at[pl.ds(row0, ROWS_PT)],
                    out_hbm.at[pl.ds(c * NPAD + row0, ROWS_PT)])


_agg_call = functools.partial(
    pl.kernel,
    out_type=jax.ShapeDtypeStruct((NC * NPAD, DH), jnp.float32),
    mesh=_mesh,
    scratch_types=[
        pltpu.VMEM((SB, K), jnp.int32),
        pltpu.VMEM((SB, K), jnp.int32),
        pltpu.VMEM((2, K, DH), jnp.float32),
        pltpu.VMEM_SHARED((NPAD, DH), jnp.float32),
        pltpu.SemaphoreType.DMA,
        pltpu.SemaphoreType.DMA,
        pltpu.SemaphoreType.DMA,
        pltpu.SemaphoreType.DMA,
    ],
)(_agg_body)


R = 400
NB = N // R


def _dinv_block(deg_ref):
    d = deg_ref[0, :, 0:1] + deg_ref[1, :, 0:1] + 1.0
    return jnp.where(d > 0, lax.rsqrt(jnp.maximum(d, 1e-12)), 0.0)


def _mm1_body(deg_ref, x_ref, w_ref, o_ref):
    dinv = _dinv_block(deg_ref)
    h = jnp.dot(x_ref[...], w_ref[...], preferred_element_type=jnp.float32)
    g = h * dinv
    o_ref[0] = g[:, :DH]
    o_ref[1] = g[:, DH:]


def _layer_body(deg_ref, agg_ref, g_ref, b_ref, w_ref, o_ref):
    dinv = _dinv_block(deg_ref)
    x0 = jnp.maximum((agg_ref[0] + g_ref[0]) * dinv + b_ref[0, :DH], 0.0)
    x1 = jnp.maximum((agg_ref[1] + g_ref[1]) * dinv + b_ref[0, DH:], 0.0)
    x = jnp.concatenate([x0, x1], axis=1)
    h = jnp.dot(x, w_ref[...], preferred_element_type=jnp.float32)
    g = h * dinv
    o_ref[0] = g[:, :DH]
    o_ref[1] = g[:, DH:]


def _final_body(deg_ref, agg_ref, g_ref, b_ref, o_ref):
    dinv = _dinv_block(deg_ref)
    x0 = (agg_ref[0] + g_ref[0]) * dinv + b_ref[0, :DH]
    x1 = (agg_ref[1] + g_ref[1]) * dinv + b_ref[0, DH:]
    o_ref[...] = jnp.concatenate([x0, x1], axis=1)


_deg_spec = pl.BlockSpec((2, R, 16), lambda j: (0, j, 0))
_half_spec = pl.BlockSpec((2, R, DH), lambda j: (0, j, 0))
_b_spec = pl.BlockSpec((1, 2 * DH), lambda j: (0, 0))


def _mm1(deg2, x, W1):
    return pl.pallas_call(
        _mm1_body,
        grid=(NB,),
        in_specs=[
            _deg_spec,
            pl.BlockSpec((R, 128), lambda j: (j, 0)),
            pl.BlockSpec((128, 2 * DH), lambda j: (0, 0)),
        ],
        out_specs=_half_spec,
        out_shape=jax.ShapeDtypeStruct((2, NPAD, DH), jnp.float32),
    )(deg2, x, W1)


def _layer(deg2, agg, g, b, W):
    return pl.pallas_call(
        _layer_body,
        grid=(NB,),
        in_specs=[
            _deg_spec,
            _half_spec,
            _half_spec,
            _b_spec,
            pl.BlockSpec((2 * DH, 2 * DH), lambda j: (0, 0)),
        ],
        out_specs=_half_spec,
        out_shape=jax.ShapeDtypeStruct((2, NPAD, DH), jnp.float32),
    )(deg2, agg, g, b, W)


def _final(deg2, agg, g, b):
    return pl.pallas_call(
        _final_body,
        grid=(NB,),
        in_specs=[_deg_spec, _half_spec, _half_spec, _b_spec],
        out_specs=pl.BlockSpec((R, 2 * DH), lambda j: (j, 0)),
        out_shape=jax.ShapeDtypeStruct((N, 2 * DH), jnp.float32),
    )(deg2, agg, g, b)


def _agg_jnp(src, dst, g):
    gs = g.reshape(NC, NPAD, DH)
    a0 = jax.ops.segment_sum(gs[0][src], dst, num_segments=NPAD)
    a1 = jax.ops.segment_sum(gs[1][src + NPAD - NPAD], dst, num_segments=NPAD)
    a1 = jax.ops.segment_sum(gs[1][src], dst, num_segments=NPAD)
    return jnp.stack([a0, a1]).reshape(NC * NPAD, DH)


def _deg_jnp(dst):
    d = jax.ops.segment_sum(jnp.ones((E,), jnp.float32), dst, num_segments=NPAD)
    z = jnp.zeros((NPAD, 16), jnp.float32)
    d16 = jnp.broadcast_to(d[:, None], (NPAD, 16))
    return jnp.concatenate([d16, z], axis=0)


DEBUG_DEG_JNP = True
DEBUG_AGG_JNP = False


def kernel(x, edge_index, W1, b1, W2, b2, W3, b3):
    src_r = edge_index[0].reshape(NS, NSB_AGG, SB, K)
    dst_r = edge_index[1].reshape(NS, NSB_AGG, SB, K)
    b1r = b1.reshape(1, 2 * DH)
    b2r = b2.reshape(1, 2 * DH)
    b3r = b3.reshape(1, 2 * DH)

    dst_d = edge_index[1].reshape(NS * NC, NSB_DEG, SB, K)
    if DEBUG_DEG_JNP:
        deg2 = _deg_jnp(edge_index[1]).reshape(2, NPAD, 16)
    else:
        deg2 = _deg_call(dst_d).reshape(2, NPAD, 16)

    g1 = _mm1(deg2, x, W1)
    if DEBUG_AGG_JNP:
        agg1 = _agg_jnp(edge_index[0], edge_index[1], g1).reshape(2, NPAD, DH)
    else:
        agg1 = _agg_call(src_r, dst_r, g1.reshape(NC * NPAD, DH)).reshape(2, NPAD, DH)
    g2 = _layer(deg2, agg1, g1, b1r, W2)
    if DEBUG_AGG_JNP:
        agg2 = _agg_jnp(edge_index[0], edge_index[1], g2).reshape(2, NPAD, DH)
    else:
        agg2 = _agg_call(src_r, dst_r, g2.reshape(NC * NPAD, DH)).reshape(2, NPAD, DH)
    g3 = _layer(deg2, agg2, g2, b2r, W3)
    if DEBUG_AGG_JNP:
        agg3 = _agg_jnp(edge_index[0], edge_index[1], g3).reshape(2, NPAD, DH)
    else:
        agg3 = _agg_call(src_r, dst_r, g3.reshape(NC * NPAD, DH)).reshape(2, NPAD, DH)
    return _final(deg2, agg3, g3, b3r)


# speedup vs baseline: 13.8612x; 1.2718x over previous
import functools
import jax
import jax.numpy as jnp
from jax import lax
from jax.experimental import pallas as pl
from jax.experimental.pallas import tpu as pltpu
from jax.experimental.pallas import tpu_sc as plsc

N = 10000
E = 320000
DH = 128
NC = 2
NS = 16
K = 80
NCHUNK_AGG = (E // NS) // K
NCHUNK_DEG = (E // (NS * NC)) // K
SB = 25
NSB_AGG = NCHUNK_AGG // SB
NSB_DEG = NCHUNK_DEG // SB
NPAD = 10240
ROWS_PT = NPAD // NS
ZR = 128

_mesh = plsc.VectorSubcoreMesh(core_axis_name="c", subcore_axis_name="s")


def _deg_body(dst_hbm, out_hbm, dst_v, buf, acc, ssem):
    c = lax.axis_index("c")
    s = lax.axis_index("s")
    zeros16 = jnp.zeros((16,), jnp.float32)
    ones16 = jnp.ones((16,), jnp.float32)

    def fillrow(r, carry):
        for k in range(DH // 16):
            buf[0, r, pl.ds(k * 16, 16)] = zeros16
            buf[1, r, pl.ds(k * 16, 16)] = ones16
        return carry
    lax.fori_loop(0, K, fillrow, 0)

    row0 = s * ROWS_PT
    for q in range(ROWS_PT // K):
        pltpu.sync_copy(buf.at[0], acc.at[pl.ds(row0 + q * K, K)])
    plsc.subcore_barrier()

    wid = s * NC + c

    def sblock(sb, carry):
        pltpu.sync_copy(dst_hbm.at[wid, sb], dst_v)

        def step(j, carry2):
            pltpu.async_copy(buf.at[1], acc.at[dst_v.at[j]], ssem, add=True)
            pltpu.make_async_copy(buf.at[1], acc.at[dst_v.at[j]], ssem).wait()
            return carry2
        lax.fori_loop(0, SB, step, 0)
        return carry
    lax.fori_loop(0, NSB_DEG, sblock, 0)

    plsc.subcore_barrier()
    pltpu.sync_copy(acc.at[pl.ds(row0, ROWS_PT)],
                    out_hbm.at[pl.ds(c * NPAD + row0, ROWS_PT)])


_deg_call = functools.partial(
    pl.kernel,
    out_type=jax.ShapeDtypeStruct((NC * NPAD, DH), jnp.float32),
    mesh=_mesh,
    scratch_types=[
        pltpu.VMEM((SB, K), jnp.int32),
        pltpu.VMEM((2, K, DH), jnp.float32),
        pltpu.VMEM_SHARED((NPAD, DH), jnp.float32),
        pltpu.SemaphoreType.DMA,
    ],
)(_deg_body)


def _agg_body(src_hbm, dst_hbm, g_hbm, out_hbm, src_v, dst_v, buf, acc,
              gsem0, gsem1, ssem0, ssem1):
    c = lax.axis_index("c")
    s = lax.axis_index("s")
    zeros16 = jnp.zeros((16,), jnp.float32)

    def zrow(r, carry):
        for k in range(DH // 16):
            buf[0, r, pl.ds(k * 16, 16)] = zeros16
        return carry
    lax.fori_loop(0, K, zrow, 0)

    row0 = s * ROWS_PT
    for q in range(ROWS_PT // K):
        pltpu.sync_copy(buf.at[0], acc.at[pl.ds(row0 + q * K, K)])

    off = c * NPAD
    plsc.subcore_barrier()

    def gather(j, b, sem):
        pltpu.async_copy(g_hbm.at[src_v.at[j]], buf.at[b], sem)

    def gwait(j, b, sem):
        pltpu.make_async_copy(g_hbm.at[src_v.at[j]], buf.at[b], sem).wait()

    def scatter(j, b, sem):
        pltpu.async_copy(buf.at[b], acc.at[dst_v.at[j]], sem, add=True)
        pltpu.make_async_copy(buf.at[b], acc.at[dst_v.at[j]], sem).wait()

    def sblock(sb, carry):
        pltpu.sync_copy(src_hbm.at[s, sb], src_v)
        pltpu.sync_copy(dst_hbm.at[s, sb], dst_v)

        def addoff(r, carry2):
            for k in range(K // 16):
                sl = pl.ds(k * 16, 16)
                src_v[r, sl] = src_v[r, sl] + off
            return carry2
        lax.fori_loop(0, SB, addoff, 0)

        gather(0, 0, gsem0)

        def pair(i, carry2):
            j0 = i * 2
            gwait(j0, 0, gsem0)
            gather(j0 + 1, 1, gsem1)
            scatter(j0, 0, ssem0)
            gwait(j0 + 1, 1, gsem1)
            gather(jnp.minimum(j0 + 2, SB - 1), 0, gsem0)
            scatter(j0 + 1, 1, ssem1)
            return carry2
        lax.fori_loop(0, SB // 2, pair, 0)

        gwait(SB - 1, 0, gsem0)
        scatter(SB - 1, 0, ssem0)
        return carry
    lax.fori_loop(0, NSB_AGG, sblock, 0)

    plsc.subcore_barrier()
    pltpu.sync_copy(acc.at[pl.ds(row0, ROWS_PT)],
                    out_hbm.at[pl.ds(c * NPAD + row0, ROWS_PT)])


_agg_call = functools.partial(
    pl.kernel,
    out_type=jax.ShapeDtypeStruct((NC * NPAD, DH), jnp.float32),
    mesh=_mesh,
    scratch_types=[
        pltpu.VMEM((SB, K), jnp.int32),
        pltpu.VMEM((SB, K), jnp.int32),
        pltpu.VMEM((2, K, DH), jnp.float32),
        pltpu.VMEM_SHARED((NPAD, DH), jnp.float32),
        pltpu.SemaphoreType.DMA,
        pltpu.SemaphoreType.DMA,
        pltpu.SemaphoreType.DMA,
        pltpu.SemaphoreType.DMA,
    ],
)(_agg_body)


R = 400
NB = N // R


def _dinv_block(deg_ref):
    d = deg_ref[0, :, 0:1] + deg_ref[1, :, 0:1] + 1.0
    return jnp.where(d > 0, lax.rsqrt(jnp.maximum(d, 1e-12)), 0.0)


def _mm1_body(deg_ref, x_ref, w_ref, o_ref):
    dinv = _dinv_block(deg_ref)
    h = jnp.dot(x_ref[...], w_ref[...], preferred_element_type=jnp.float32)
    g = h * dinv
    o_ref[0] = g[:, :DH]
    o_ref[1] = g[:, DH:]


def _layer_body(deg_ref, agg_ref, g_ref, b_ref, w_ref, o_ref):
    dinv = _dinv_block(deg_ref)
    x0 = jnp.maximum((agg_ref[0] + g_ref[0]) * dinv + b_ref[0, :DH], 0.0)
    x1 = jnp.maximum((agg_ref[1] + g_ref[1]) * dinv + b_ref[0, DH:], 0.0)
    x = jnp.concatenate([x0, x1], axis=1)
    h = jnp.dot(x, w_ref[...], preferred_element_type=jnp.float32)
    g = h * dinv
    o_ref[0] = g[:, :DH]
    o_ref[1] = g[:, DH:]


def _final_body(deg_ref, agg_ref, g_ref, b_ref, o_ref):
    dinv = _dinv_block(deg_ref)
    x0 = (agg_ref[0] + g_ref[0]) * dinv + b_ref[0, :DH]
    x1 = (agg_ref[1] + g_ref[1]) * dinv + b_ref[0, DH:]
    o_ref[...] = jnp.concatenate([x0, x1], axis=1)


_deg_spec = pl.BlockSpec((2, R, DH), lambda j: (0, j, 0))
_half_spec = pl.BlockSpec((2, R, DH), lambda j: (0, j, 0))
_b_spec = pl.BlockSpec((1, 2 * DH), lambda j: (0, 0))


def _mm1(deg2, x, W1):
    return pl.pallas_call(
        _mm1_body,
        grid=(NB,),
        in_specs=[
            _deg_spec,
            pl.BlockSpec((R, 128), lambda j: (j, 0)),
            pl.BlockSpec((128, 2 * DH), lambda j: (0, 0)),
        ],
        out_specs=_half_spec,
        out_shape=jax.ShapeDtypeStruct((2, NPAD, DH), jnp.float32),
    )(deg2, x, W1)


def _layer(deg2, agg, g, b, W):
    return pl.pallas_call(
        _layer_body,
        grid=(NB,),
        in_specs=[
            _deg_spec,
            _half_spec,
            _half_spec,
            _b_spec,
            pl.BlockSpec((2 * DH, 2 * DH), lambda j: (0, 0)),
        ],
        out_specs=_half_spec,
        out_shape=jax.ShapeDtypeStruct((2, NPAD, DH), jnp.float32),
    )(deg2, agg, g, b, W)


def _final(deg2, agg, g, b):
    return pl.pallas_call(
        _final_body,
        grid=(NB,),
        in_specs=[_deg_spec, _half_spec, _half_spec, _b_spec],
        out_specs=pl.BlockSpec((R, 2 * DH), lambda j: (j, 0)),
        out_shape=jax.ShapeDtypeStruct((N, 2 * DH), jnp.float32),
    )(deg2, agg, g, b)


def kernel(x, edge_index, W1, b1, W2, b2, W3, b3):
    src_r = edge_index[0].reshape(NS, NSB_AGG, SB, K)
    dst_r = edge_index[1].reshape(NS, NSB_AGG, SB, K)
    b1r = b1.reshape(1, 2 * DH)
    b2r = b2.reshape(1, 2 * DH)
    b3r = b3.reshape(1, 2 * DH)

    dst_d = edge_index[1].reshape(NS * NC, NSB_DEG, SB, K)
    deg2 = _deg_call(dst_d).reshape(2, NPAD, DH)

    g1 = _mm1(deg2, x, W1)
    agg1 = _agg_call(src_r, dst_r, g1.reshape(NC * NPAD, DH)).reshape(2, NPAD, DH)
    g2 = _layer(deg2, agg1, g1, b1r, W2)
    agg2 = _agg_call(src_r, dst_r, g2.reshape(NC * NPAD, DH)).reshape(2, NPAD, DH)
    g3 = _layer(deg2, agg2, g2, b2r, W3)
    agg3 = _agg_call(src_r, dst_r, g3.reshape(NC * NPAD, DH)).reshape(2, NPAD, DH)
    return _final(deg2, agg3, g3, b3r)
